# Initial kernel scaffold; baseline (speedup 1.0000x reference)
import jax
import jax.numpy as jnp
from jax.experimental import pallas as pl


def kernel(x, levels):
    raise NotImplementedError("write your pallas kernel here")


# speedup vs baseline: 393.2801x; 393.2801x over previous
import functools

import jax
import jax.numpy as jnp
from jax import lax
from jax.experimental import pallas as pl
from jax.experimental.pallas import tpu as pltpu
from jax.experimental.pallas import tpu_sc as plsc

GROUP = 64
NLEV = 16
NELEM = 2048 * 4096
NGROUPS = NELEM // GROUP
NWORKERS = 32
GPW = NGROUPS // NWORKERS
CHUNK = 256
NCHUNK = GPW // CHUNK

_HEAP = (7, 3, 11, 1, 5, 9, 13, 0, 2, 4, 6, 8, 10, 12, 14, 14)


def _sc_body(x_hbm, lv_hbm, out_hbm, xbuf, lvbuf, bbuf, obuf):
    info = plsc.get_sparse_core_info()
    nc = info.num_cores
    wid = lax.axis_index("s") * nc + lax.axis_index("c")

    ii = lax.iota(jnp.int32, 16)
    zero = ii * 0
    one = zero + 1
    two = zero + 2
    d = (
        jnp.where(ii >= 1, one, zero)
        + jnp.where(ii >= 3, one, zero)
        + jnp.where(ii >= 7, one, zero)
        + jnp.where(ii >= 15, one, zero)
    )
    heap = (ii - lax.shift_left(one, d) + 1) * lax.shift_right_logical(
        zero + 16, d
    ) + lax.shift_right_logical(zero + 8, d) - 1
    heap = jnp.maximum(heap, zero)

    def chunk_body(c, carry):
        base_g = wid * GPW + c * CHUNK
        pltpu.sync_copy(x_hbm.at[pl.ds(base_g * GROUP, CHUNK * GROUP)], xbuf)
        pltpu.sync_copy(lv_hbm.at[pl.ds(base_g * NLEV, CHUNK * NLEV)], lvbuf)

        def border_body(g, carry2):
            b16 = jnp.broadcast_to(g * NLEV, (16,)).astype(jnp.int32)
            pa = b16 + heap
            lo = plsc.load_gather(lvbuf, [pa])
            hi = plsc.load_gather(lvbuf, [pa + one])
            bbuf[pl.ds(g * NLEV, NLEV)] = (lo + hi) * 0.5
            return carry2

        lax.fori_loop(0, CHUNK, border_body, 0, unroll=4)

        def search_body(g, carry2):
            b16 = jnp.broadcast_to(g * NLEV, (16,)).astype(jnp.int32)
            for k in range(GROUP // 16):
                xv = xbuf[pl.ds(g * GROUP + k * 16, 16)]
                pos = zero
                for _ in range(4):
                    bv = plsc.load_gather(bbuf, [b16 + pos])
                    step = jnp.where(xv > bv, two, one)
                    pos = pos + pos + step
                idx = b16 + pos - 15
                obuf[pl.ds(g * GROUP + k * 16, 16)] = plsc.load_gather(
                    lvbuf, [idx]
                )
            return carry2

        lax.fori_loop(0, CHUNK, search_body, 0, unroll=2)

        pltpu.sync_copy(obuf, out_hbm.at[pl.ds(base_g * GROUP, CHUNK * GROUP)])
        return carry

    lax.fori_loop(0, NCHUNK, chunk_body, 0)


@jax.jit
def _quantize(xf, lf):
    mesh = plsc.VectorSubcoreMesh(core_axis_name="c", subcore_axis_name="s")
    return pl.kernel(
        _sc_body,
        out_type=jax.ShapeDtypeStruct((NELEM,), jnp.float32),
        mesh=mesh,
        scratch_types=[
            pltpu.VMEM((CHUNK * GROUP,), jnp.float32),
            pltpu.VMEM((CHUNK * NLEV,), jnp.float32),
            pltpu.VMEM((CHUNK * NLEV,), jnp.float32),
            pltpu.VMEM((CHUNK * GROUP,), jnp.float32),
        ],
        compiler_params=pltpu.CompilerParams(needs_layout_passes=False),
    )(xf, lf)


def kernel(x, levels):
    out = _quantize(x.reshape(-1), levels.reshape(-1))
    return out.reshape(x.shape)


# speedup vs baseline: 1257.7758x; 3.1982x over previous
import functools

import jax
import jax.numpy as jnp
from jax import lax
from jax.experimental import pallas as pl
from jax.experimental.pallas import tpu as pltpu
from jax.experimental.pallas import tpu_sc as plsc

GROUP = 64
NLEV = 16
NELEM = 2048 * 4096
NGROUPS = NELEM // GROUP
NWORKERS = 32
GPW = NGROUPS // NWORKERS
CHUNK = 256
NCHUNK = GPW // CHUNK

_HEAP = (7, 3, 11, 1, 5, 9, 13, 0, 2, 4, 6, 8, 10, 12, 14, 14)


def _sc_body(x_hbm, lv_hbm, out_hbm, xbuf, lvbuf, bbuf, obuf):
    info = plsc.get_sparse_core_info()
    nc = info.num_cores
    wid = lax.axis_index("s") * nc + lax.axis_index("c")

    ii = lax.iota(jnp.int32, 16)
    zero = ii * 0
    one = zero + 1
    two = zero + 2
    d = (
        jnp.where(ii >= 1, one, zero)
        + jnp.where(ii >= 3, one, zero)
        + jnp.where(ii >= 7, one, zero)
        + jnp.where(ii >= 15, one, zero)
    )
    heap = (ii - lax.shift_left(one, d) + 1) * lax.shift_right_logical(
        zero + 16, d
    ) + lax.shift_right_logical(zero + 8, d) - 1
    heap = jnp.maximum(heap, zero)

    def chunk_body(c, carry):
        base_g = wid * GPW + c * CHUNK
        pltpu.sync_copy(x_hbm.at[pl.ds(base_g * GROUP, CHUNK * GROUP)], xbuf)
        pltpu.sync_copy(lv_hbm.at[pl.ds(base_g * NLEV, CHUNK * NLEV)], lvbuf)

        @plsc.parallel_loop(0, CHUNK, unroll=8)
        def _border(g):
            b16 = jnp.broadcast_to(g * NLEV, (16,)).astype(jnp.int32)
            pa = b16 + heap
            lo = plsc.load_gather(lvbuf, [pa])
            hi = plsc.load_gather(lvbuf, [pa + one])
            bbuf[pl.ds(g * NLEV, NLEV)] = (lo + hi) * 0.5

        @plsc.parallel_loop(0, CHUNK * (GROUP // 16), unroll=8)
        def _search(i):
            b16 = jnp.broadcast_to((i >> 2) * NLEV, (16,)).astype(jnp.int32)
            xv = xbuf[pl.ds(i * 16, 16)]
            pos = zero
            for _ in range(4):
                bv = plsc.load_gather(bbuf, [b16 + pos])
                step = jnp.where(xv > bv, two, one)
                pos = pos + pos + step
            idx = b16 + pos - 15
            obuf[pl.ds(i * 16, 16)] = plsc.load_gather(lvbuf, [idx])

        pltpu.sync_copy(obuf, out_hbm.at[pl.ds(base_g * GROUP, CHUNK * GROUP)])
        return carry

    lax.fori_loop(0, NCHUNK, chunk_body, 0)


@jax.jit
def _quantize(xf, lf):
    mesh = plsc.VectorSubcoreMesh(core_axis_name="c", subcore_axis_name="s")
    return pl.kernel(
        _sc_body,
        out_type=jax.ShapeDtypeStruct((NELEM,), jnp.float32),
        mesh=mesh,
        scratch_types=[
            pltpu.VMEM((CHUNK * GROUP,), jnp.float32),
            pltpu.VMEM((CHUNK * NLEV,), jnp.float32),
            pltpu.VMEM((CHUNK * NLEV,), jnp.float32),
            pltpu.VMEM((CHUNK * GROUP,), jnp.float32),
        ],
        compiler_params=pltpu.CompilerParams(needs_layout_passes=False),
    )(xf, lf)


def kernel(x, levels):
    out = _quantize(x.reshape(-1), levels.reshape(-1))
    return out.reshape(x.shape)


# speedup vs baseline: 1548.0453x; 1.2308x over previous
import functools

import jax
import jax.numpy as jnp
from jax import lax
from jax.experimental import pallas as pl
from jax.experimental.pallas import tpu as pltpu
from jax.experimental.pallas import tpu_sc as plsc

GROUP = 64
NLEV = 16
NELEM = 2048 * 4096
NGROUPS = NELEM // GROUP
NWORKERS = 32
GPW = NGROUPS // NWORKERS
CHUNK = 256
NCHUNK = GPW // CHUNK

_HEAP = (7, 3, 11, 1, 5, 9, 13, 0, 2, 4, 6, 8, 10, 12, 14, 14)


def _sc_body(
    x_hbm,
    lv_hbm,
    out_hbm,
    xbuf0,
    xbuf1,
    lvbuf0,
    lvbuf1,
    obuf0,
    obuf1,
    bbuf,
    sin0,
    sin1,
    sout0,
    sout1,
):
    info = plsc.get_sparse_core_info()
    nc = info.num_cores
    wid = lax.axis_index("s") * nc + lax.axis_index("c")

    ii = lax.iota(jnp.int32, 16)
    zero = ii * 0
    one = zero + 1
    two = zero + 2
    d = (
        jnp.where(ii >= 1, one, zero)
        + jnp.where(ii >= 3, one, zero)
        + jnp.where(ii >= 7, one, zero)
        + jnp.where(ii >= 15, one, zero)
    )
    heap = (ii - lax.shift_left(one, d) + 1) * lax.shift_right_logical(
        zero + 16, d
    ) + lax.shift_right_logical(zero + 8, d) - 1
    heap = jnp.maximum(heap, zero)

    xbufs = (xbuf0, xbuf1)
    lvbufs = (lvbuf0, lvbuf1)
    obufs = (obuf0, obuf1)
    sins = (sin0, sin1)
    souts = (sout0, sout1)

    def start_in(c, half):
        base_g = wid * GPW + c * CHUNK
        pltpu.async_copy(
            x_hbm.at[pl.ds(base_g * GROUP, CHUNK * GROUP)], xbufs[half],
            sins[half],
        )
        pltpu.async_copy(
            lv_hbm.at[pl.ds(base_g * NLEV, CHUNK * NLEV)], lvbufs[half],
            sins[half],
        )

    def wait_in(half):
        pltpu.make_async_copy(
            x_hbm.at[pl.ds(0, CHUNK * GROUP)], xbufs[half], sins[half]
        ).wait()
        pltpu.make_async_copy(
            lv_hbm.at[pl.ds(0, CHUNK * NLEV)], lvbufs[half], sins[half]
        ).wait()

    def wait_out(half):
        pltpu.make_async_copy(
            x_hbm.at[pl.ds(0, CHUNK * GROUP)], obufs[half], souts[half]
        ).wait()

    def compute(half):
        xbuf, lvbuf, obuf = xbufs[half], lvbufs[half], obufs[half]

        @plsc.parallel_loop(0, CHUNK, unroll=8)
        def _border(g):
            b16 = jnp.broadcast_to(g * NLEV, (16,)).astype(jnp.int32)
            pa = b16 + heap
            lo = plsc.load_gather(lvbuf, [pa])
            hi = plsc.load_gather(lvbuf, [pa + one])
            bbuf[pl.ds(g * NLEV, NLEV)] = (lo + hi) * 0.5

        @plsc.parallel_loop(0, CHUNK * (GROUP // 16), unroll=8)
        def _search(i):
            b16 = jnp.broadcast_to((i >> 2) * NLEV, (16,)).astype(jnp.int32)
            xv = xbuf[pl.ds(i * 16, 16)]
            pos = zero
            for _ in range(4):
                bv = plsc.load_gather(bbuf, [b16 + pos])
                step = jnp.where(xv > bv, two, one)
                pos = pos + pos + step
            idx = b16 + pos - 15
            obuf[pl.ds(i * 16, 16)] = plsc.load_gather(lvbuf, [idx])

    start_in(0, 0)

    def pair_body(k, carry):
        for half in range(2):
            c = 2 * k + half
            wait_in(half)

            @pl.when(c + 1 < NCHUNK)
            def _():
                start_in(c + 1, 1 - half)

            @pl.when(k > 0)
            def _():
                wait_out(half)

            compute(half)

            base_g = wid * GPW + c * CHUNK
            pltpu.async_copy(
                obufs[half],
                out_hbm.at[pl.ds(base_g * GROUP, CHUNK * GROUP)],
                souts[half],
            )
        return carry

    lax.fori_loop(0, NCHUNK // 2, pair_body, 0)
    wait_out(0)
    wait_out(1)


@jax.jit
def _quantize(xf, lf):
    mesh = plsc.VectorSubcoreMesh(core_axis_name="c", subcore_axis_name="s")
    return pl.kernel(
        _sc_body,
        out_type=jax.ShapeDtypeStruct((NELEM,), jnp.float32),
        mesh=mesh,
        scratch_types=[
            pltpu.VMEM((CHUNK * GROUP,), jnp.float32),
            pltpu.VMEM((CHUNK * GROUP,), jnp.float32),
            pltpu.VMEM((CHUNK * NLEV,), jnp.float32),
            pltpu.VMEM((CHUNK * NLEV,), jnp.float32),
            pltpu.VMEM((CHUNK * GROUP,), jnp.float32),
            pltpu.VMEM((CHUNK * GROUP,), jnp.float32),
            pltpu.VMEM((CHUNK * NLEV,), jnp.float32),
            pltpu.SemaphoreType.DMA,
            pltpu.SemaphoreType.DMA,
            pltpu.SemaphoreType.DMA,
            pltpu.SemaphoreType.DMA,
        ],
        compiler_params=pltpu.CompilerParams(needs_layout_passes=False),
    )(xf, lf)


def kernel(x, levels):
    out = _quantize(x.reshape(-1), levels.reshape(-1))
    return out.reshape(x.shape)


# speedup vs baseline: 1672.9608x; 1.0807x over previous
import functools

import jax
import jax.numpy as jnp
from jax import lax
from jax.experimental import pallas as pl
from jax.experimental.pallas import tpu as pltpu
from jax.experimental.pallas import tpu_sc as plsc

ROWS, COLS = 2048, 4096
GROUP = 64
NLEV = 16
NELEM = ROWS * COLS
NGROUPS = NELEM // GROUP
NWORKERS = 32
CROWS = 8
CELEM = CROWS * COLS
CGROUP = CELEM // GROUP
NCHUNK = (ROWS // CROWS) // NWORKERS
HELEM = CELEM // 2


def _sc_body(
    x_hbm,
    lv_hbm,
    out_hbm,
    xbuf0,
    xbuf1,
    lvbuf0,
    lvbuf1,
    bbuf,
    obufa,
    obufb,
    sin0,
    sin1,
    souta,
    soutb,
):
    info = plsc.get_sparse_core_info()
    nc = info.num_cores
    wid = lax.axis_index("s") * nc + lax.axis_index("c")

    ii = lax.iota(jnp.int32, 16)
    zero = ii * 0
    one = zero + 1
    two = zero + 2
    d = (
        jnp.where(ii >= 1, one, zero)
        + jnp.where(ii >= 3, one, zero)
        + jnp.where(ii >= 7, one, zero)
        + jnp.where(ii >= 15, one, zero)
    )
    heap = (ii - lax.shift_left(one, d) + 1) * lax.shift_right_logical(
        zero + 16, d
    ) + lax.shift_right_logical(zero + 8, d) - 1
    heap = jnp.maximum(heap, zero)

    xbufs = (xbuf0, xbuf1)
    lvbufs = (lvbuf0, lvbuf1)
    sins = (sin0, sin1)

    def start_in(c, half):
        tr = wid * NCHUNK + c
        pltpu.async_copy(
            x_hbm.at[pl.ds(tr * CROWS, CROWS), :], xbufs[half], sins[half]
        )
        pltpu.async_copy(
            lv_hbm.at[pl.ds(tr * CGROUP * NLEV, CGROUP * NLEV)],
            lvbufs[half],
            sins[half],
        )

    def wait_in(half):
        pltpu.make_async_copy(
            x_hbm.at[pl.ds(0, CROWS), :], xbufs[half], sins[half]
        ).wait()
        pltpu.make_async_copy(
            lv_hbm.at[pl.ds(0, CGROUP * NLEV)], lvbufs[half], sins[half]
        ).wait()

    def wait_out(obuf, sout):
        pltpu.make_async_copy(
            lv_hbm.at[pl.ds(0, HELEM)], obuf, sout
        ).wait()

    def compute_half(half, part):
        xbuf, lvbuf = xbufs[half], lvbufs[half]
        obuf = obufa if part == 0 else obufb

        @plsc.parallel_loop(part * (HELEM // 16), (part + 1) * (HELEM // 16),
                            unroll=8)
        def _search(i):
            f = i * 16
            b16 = jnp.broadcast_to((i >> 2) * NLEV, (16,)).astype(jnp.int32)
            xv = xbuf[f >> 12, pl.ds(f & (COLS - 1), 16)]
            pos = zero
            for _ in range(4):
                bv = plsc.load_gather(bbuf, [b16 + pos])
                step = jnp.where(xv > bv, two, one)
                pos = pos + pos + step
            idx = b16 + pos - 15
            obuf[pl.ds(f - part * HELEM, 16)] = plsc.load_gather(lvbuf, [idx])

    start_in(0, 0)

    def pair_body(k, carry):
        for half in range(2):
            c = 2 * k + half
            tr = wid * NCHUNK + c
            wait_in(half)

            @pl.when(c + 1 < NCHUNK)
            def _():
                start_in(c + 1, 1 - half)

            lvbuf = lvbufs[half]

            @plsc.parallel_loop(0, CGROUP, unroll=8)
            def _border(g):
                b16 = jnp.broadcast_to(g * NLEV, (16,)).astype(jnp.int32)
                pa = b16 + heap
                lo = plsc.load_gather(lvbuf, [pa])
                hi = plsc.load_gather(lvbuf, [pa + one])
                bbuf[pl.ds(g * NLEV, NLEV)] = (lo + hi) * 0.5

            @pl.when(c > 0)
            def _():
                wait_out(obufa, souta)

            compute_half(half, 0)
            pltpu.async_copy(
                obufa, out_hbm.at[pl.ds(tr * CELEM, HELEM)], souta
            )

            @pl.when(c > 0)
            def _():
                wait_out(obufb, soutb)

            compute_half(half, 1)
            pltpu.async_copy(
                obufb, out_hbm.at[pl.ds(tr * CELEM + HELEM, HELEM)], soutb
            )
        return carry

    lax.fori_loop(0, NCHUNK // 2, pair_body, 0)
    wait_out(obufa, souta)
    wait_out(obufb, soutb)


@jax.jit
def _quantize(x, lf):
    mesh = plsc.VectorSubcoreMesh(core_axis_name="c", subcore_axis_name="s")
    return pl.kernel(
        _sc_body,
        out_type=jax.ShapeDtypeStruct((NELEM,), jnp.float32),
        mesh=mesh,
        scratch_types=[
            pltpu.VMEM((CROWS, COLS), jnp.float32),
            pltpu.VMEM((CROWS, COLS), jnp.float32),
            pltpu.VMEM((CGROUP * NLEV,), jnp.float32),
            pltpu.VMEM((CGROUP * NLEV,), jnp.float32),
            pltpu.VMEM((CGROUP * NLEV,), jnp.float32),
            pltpu.VMEM((HELEM,), jnp.float32),
            pltpu.VMEM((HELEM,), jnp.float32),
            pltpu.SemaphoreType.DMA,
            pltpu.SemaphoreType.DMA,
            pltpu.SemaphoreType.DMA,
            pltpu.SemaphoreType.DMA,
        ],
        compiler_params=pltpu.CompilerParams(
            needs_layout_passes=False, use_tc_tiling_on_sc=True
        ),
    )(x, lf)


def kernel(x, levels):
    out = _quantize(x, levels.reshape(-1))
    return out.reshape(x.shape)
